# baseline (speedup 1.0000x reference)
import math

import numpy as np
import jax
import jax.numpy as jnp
from jax.experimental import pallas as pl
from jax.experimental.pallas import tpu as pltpu

B, M, L = 128, 32, 256
PROJ, CNN_OUT, DM, NH, NLAYERS = 32, 32, 64, 4, 2
E, TOPK, EH, MOE_OUT = 64, 2, 128, 64
PREDH, FAILH = 8, 3
K = 3
DILS = (1, 2, 4)
NB = 4
NBM = NB * M
R = NBM * L


def _pe(length, d):
    pe = np.zeros((length, d), dtype=np.float32)
    pos = np.arange(length, dtype=np.float32)[:, None]
    div = np.exp(np.arange(0, d, 2, dtype=np.float32) * (-math.log(10000.0) / d))
    pe[:, 0::2] = np.sin(pos * div)
    pe[:, 1::2] = np.cos(pos * div)
    return pe


def _lnk(x, g, b, eps=1e-5):
    mu = jnp.mean(x, -1, keepdims=True)
    var = jnp.mean((x - mu) ** 2, -1, keepdims=True)
    return (x - mu) * jax.lax.rsqrt(var + eps) * g + b


def _shift_rows(h, s, iota_l):
    if s == 0:
        return h
    top = jnp.zeros((s, h.shape[1]), h.dtype)
    sh = jnp.concatenate([top, h[: h.shape[0] - s, :]], axis=0)
    return jnp.where(iota_l >= s, sh, 0.0)


def _encoder_body(x_ref, mask_ref, maskr_ref, revw_ref, revb_ref, pw_ref,
                  pepb_ref, wc0_ref, cb0_ref, wc1_ref, cb1_ref, wc2_ref,
                  cb2_ref, encg_ref, encb_ref, poolw_ref, poolb_ref, pos_ref,
                  tw_ref, outg_ref, outb_ref, hwlf_ref, hwt_ref, hb_ref,
                  gw_ref, gb_ref,
                  g_ref, logits_ref, head_ref, sd_ref):
    x3 = x_ref[...]
    mu = jnp.mean(x3, -1, keepdims=True)
    var = jnp.mean((x3 - mu) ** 2, -1, keepdims=True)
    sd = jnp.sqrt(var + 1e-5)
    sd_ref[...] = sd
    xn3 = (x3 - mu) / sd * revw_ref[...] + revb_ref[...]
    xn2 = xn3.reshape(NBM, L)

    h0 = xn2[:, :, None] * pw_ref[...][None, :, :] + pepb_ref[...][None, :, :]
    h = h0.reshape(R, PROJ)
    iota_l = jax.lax.broadcasted_iota(jnp.int32, (R, 1), 0) % L

    for wc_ref, cb_ref, dil in ((wc0_ref, cb0_ref, 1), (wc1_ref, cb1_ref, 2),
                                (wc2_ref, cb2_ref, 4)):
        a2 = _shift_rows(h, 2 * dil, iota_l)
        a1 = _shift_rows(h, dil, iota_l)
        cat = jnp.concatenate([a2, a1, h], axis=1)
        h = jnp.maximum(jnp.dot(cat, wc_ref[...],
                                preferred_element_type=jnp.float32)
                        + cb_ref[...], 0.0)

    h = _lnk(h, encg_ref[...], encb_ref[...])
    h3 = h.reshape(NBM, L, CNN_OUT)
    maskr = maskr_ref[...]
    pooled = jnp.mean(h3, axis=1) * maskr
    lastf = h3[:, L - 1, :] * maskr

    t = (jnp.dot(pooled, poolw_ref[...], preferred_element_type=jnp.float32)
         + poolb_ref[...] + pos_ref[...])

    tw = tw_ref[...]
    maskv = mask_ref[0]
    g_rows, hp_rows = [], []
    for s in range(NB):
        ts = t[s * M:(s + 1) * M]
        mrow = maskv[s:s + 1, :]
        bias = (mrow - 1.0) * 1e9
        off = 0
        for _ in range(NLAYERS):
            ln1g = tw[off + 0:off + 1, :DM]; ln1b = tw[off + 1:off + 2, :DM]
            wq = tw[off + 2:off + 2 + DM, :DM];   bq = tw[off + 66:off + 67, :DM]
            wk = tw[off + 67:off + 67 + DM, :DM]; bk = tw[off + 131:off + 132, :DM]
            wv = tw[off + 132:off + 132 + DM, :DM]; bv = tw[off + 196:off + 197, :DM]
            wo = tw[off + 197:off + 197 + DM, :DM]; bo = tw[off + 261:off + 262, :DM]
            ln2g = tw[off + 262:off + 263, :DM]; ln2b = tw[off + 263:off + 264, :DM]
            ff1w = tw[off + 264:off + 264 + DM, :]
            ff1b = tw[off + 328:off + 329, :]
            ff2w = tw[off + 329:off + 329 + 2 * DM, :DM]
            ff2b = tw[off + 457:off + 458, :DM]
            off += 458

            a = _lnk(ts, ln1g, ln1b)
            q = jnp.dot(a, wq, preferred_element_type=jnp.float32) + bq
            k = jnp.dot(a, wk, preferred_element_type=jnp.float32) + bk
            v = jnp.dot(a, wv, preferred_element_type=jnp.float32) + bv
            ohs = []
            dh = DM // NH
            for hd in range(NH):
                sl = slice(hd * dh, (hd + 1) * dh)
                sc = jax.lax.dot_general(
                    q[:, sl], k[:, sl], (((1,), (1,)), ((), ())),
                    preferred_element_type=jnp.float32) * (1.0 / math.sqrt(dh))
                sc = sc + bias
                pa = jax.nn.softmax(sc, axis=-1)
                ohs.append(jnp.dot(pa, v[:, sl],
                                   preferred_element_type=jnp.float32))
            o = jnp.dot(jnp.concatenate(ohs, axis=1), wo,
                        preferred_element_type=jnp.float32) + bo
            ts = ts + o
            hh = _lnk(ts, ln2g, ln2b)
            ts = ts + jnp.dot(
                jnp.maximum(jnp.dot(hh, ff1w,
                                    preferred_element_type=jnp.float32) + ff1b,
                            0.0),
                ff2w, preferred_element_type=jnp.float32) + ff2b

        tl = _lnk(ts, outg_ref[...], outb_ref[...])
        mcol = maskr[s * M:(s + 1) * M]
        tl = tl * mcol
        cnt = jnp.maximum(jnp.sum(mcol), 1.0)
        g_rows.append(jnp.sum(tl, axis=0, keepdims=True) / cnt)
        lf = lastf[s * M:(s + 1) * M]
        hp = (jnp.dot(lf, hwlf_ref[...], preferred_element_type=jnp.float32)
              + jnp.dot(tl, hwt_ref[...], preferred_element_type=jnp.float32)
              + hb_ref[...])
        hp_rows.append(hp.reshape(1, M, PREDH + 1))

    g = jnp.concatenate(g_rows, axis=0)
    g_ref[...] = g.reshape(1, NB, DM)
    logits_ref[...] = (jnp.dot(g, gw_ref[...],
                               preferred_element_type=jnp.float32)
                       + gb_ref[...]).reshape(1, NB, 3 * E)
    head_ref[...] = jnp.concatenate(hp_rows, axis=0)


def _gate_weights(v, iota64):
    m1 = jnp.max(v, -1, keepdims=True)
    i1 = jnp.min(jnp.where(v == m1, iota64, E), -1, keepdims=True)
    sel1 = iota64 == i1
    v2 = jnp.where(sel1, -1e30, v)
    m2 = jnp.max(v2, -1, keepdims=True)
    i2 = jnp.min(jnp.where(v2 == m2, iota64, E), -1, keepdims=True)
    sel2 = iota64 == i2
    g1 = 1.0 / (1.0 + jnp.exp(m2 - m1))
    g2 = 1.0 - g1
    return jnp.where(sel1, g1, 0.0) + jnp.where(sel2, g2, 0.0)


def _moe_body(g_ref, logits_ref, head_ref, sd_ref, last_ref, w1_ref, b1_ref,
              w2_ref, b2_ref, exp_ref, pwm_ref, pb_ref, failw_ref, failb_ref,
              rwm_ref, pred_ref, fail_ref, rca_ref):
    gg = g_ref[...]
    h1 = jnp.maximum(jnp.dot(gg, w1_ref[...],
                             preferred_element_type=jnp.float32)
                     + b1_ref[...], 0.0)
    lg = logits_ref[...]
    iota64 = jax.lax.broadcasted_iota(jnp.int32, (B, E), 1)
    moes = []
    for gi in range(3):
        w = _gate_weights(lg[:, gi * E:(gi + 1) * E], iota64)
        wexp = jnp.dot(w, exp_ref[...],
                       preferred_element_type=jnp.float32)
        moe = (jnp.dot(h1 * wexp, w2_ref[...],
                       preferred_element_type=jnp.float32)
               + jnp.dot(w, b2_ref[...], preferred_element_type=jnp.float32))
        moes.append(moe)
    moe_f, moe_l, moe_r = moes

    hp = head_ref[...]
    predm = jnp.dot(moe_f, pwm_ref[...],
                    preferred_element_type=jnp.float32) + pb_ref[...]
    pred_delta = hp[:, :, 0:PREDH] + predm[:, None, :]
    pred_ref[...] = pred_delta * sd_ref[...] + last_ref[...]
    fail_ref[...] = (jnp.dot(moe_l, failw_ref[...],
                             preferred_element_type=jnp.float32)
                     + failb_ref[...])
    rcam = jnp.dot(moe_r, rwm_ref[...], preferred_element_type=jnp.float32)
    rca_ref[...] = hp[:, :, PREDH:PREDH + 1] + rcam[:, None, :]


def _const_spec(shape):
    nd = len(shape)
    return pl.BlockSpec(shape, lambda i, _n=nd: (0,) * _n)


@jax.jit
def kernel(x_features_orig_scale, sensor_mask, last_known_values_orig, params):
    p = params
    x = x_features_orig_scale
    mask = sensor_mask

    pepb = jnp.asarray(_pe(L, PROJ)) + p['proj_b'][None, :]
    pw = jnp.broadcast_to(p['proj_W'], (L, PROJ))
    wcs = [p['conv%d_W' % i].transpose(2, 1, 0).reshape(K * PROJ, -1)
           for i in range(3)]
    cbs = [p['conv%d_b' % i][None, :] for i in range(3)]
    maskr = mask.reshape(B * M, 1)
    pos_t = jnp.tile(p['pos'][0], (NB, 1))

    tw_rows = []
    for l in range(NLAYERS):
        pr = 't%d_' % l
        def pad(a):
            a = a if a.ndim == 2 else a[None, :]
            return jnp.pad(a, ((0, 0), (0, 2 * DM - a.shape[1])))
        tw_rows += [pad(p[pr + 'ln1_g']), pad(p[pr + 'ln1_b']),
                    pad(p[pr + 'Wq']), pad(p[pr + 'bq']),
                    pad(p[pr + 'Wk']), pad(p[pr + 'bk']),
                    pad(p[pr + 'Wv']), pad(p[pr + 'bv']),
                    pad(p[pr + 'Wo']), pad(p[pr + 'bo']),
                    pad(p[pr + 'ln2_g']), pad(p[pr + 'ln2_b']),
                    p[pr + 'ff1_W'], p[pr + 'ff1_b'][None, :],
                    pad(p[pr + 'ff2_W']), pad(p[pr + 'ff2_b'])]
    tw = jnp.concatenate(tw_rows, axis=0)

    hwlf = jnp.concatenate([p['pred_W'][:CNN_OUT], p['rca_W'][:CNN_OUT]], 1)
    hwt = jnp.concatenate([p['pred_W'][CNN_OUT:CNN_OUT + DM],
                           p['rca_W'][CNN_OUT:CNN_OUT + DM]], 1)
    hb = jnp.concatenate([p['pred_b'], p['rca_b']])[None, :]
    gw = jnp.concatenate([p['gf_W'], p['gl_W'], p['gr_W']], axis=1)
    gb = jnp.concatenate([p['gf_b'], p['gl_b'], p['gr_b']])[None, :]

    w1cat = p['e_W1'].transpose(1, 0, 2).reshape(DM, E * EH)
    b1cat = p['e_b1'].reshape(1, E * EH)
    w2stack = p['e_W2'].reshape(E * EH, MOE_OUT)
    b2mat = p['e_b2']
    expand = jnp.asarray(np.repeat(np.eye(E, dtype=np.float32), EH, axis=1))
    pwm = p['pred_W'][CNN_OUT + DM:]
    rwm = p['rca_W'][CNN_OUT + DM:]
    last3 = last_known_values_orig[:, :, None]

    grid = B // NB
    mask3 = mask.reshape(grid, NB, M)
    g3, logits3, head_part, sd = pl.pallas_call(
        _encoder_body,
        grid=(grid,),
        in_specs=[
            pl.BlockSpec((NB, M, L), lambda i: (i, 0, 0)),
            pl.BlockSpec((1, NB, M), lambda i: (i, 0, 0)),
            pl.BlockSpec((NBM, 1), lambda i: (i, 0)),
            _const_spec((1, M, 1)), _const_spec((1, M, 1)),
            _const_spec((L, PROJ)), _const_spec((L, PROJ)),
            _const_spec((K * PROJ, PROJ)), _const_spec((1, PROJ)),
            _const_spec((K * PROJ, PROJ)), _const_spec((1, PROJ)),
            _const_spec((K * PROJ, CNN_OUT)), _const_spec((1, CNN_OUT)),
            _const_spec((1, CNN_OUT)), _const_spec((1, CNN_OUT)),
            _const_spec((CNN_OUT, DM)), _const_spec((1, DM)),
            _const_spec((NBM, DM)),
            _const_spec(tuple(tw.shape)),
            _const_spec((1, DM)), _const_spec((1, DM)),
            _const_spec((CNN_OUT, PREDH + 1)), _const_spec((DM, PREDH + 1)),
            _const_spec((1, PREDH + 1)),
            _const_spec((DM, 3 * E)), _const_spec((1, 3 * E)),
        ],
        out_specs=[
            pl.BlockSpec((1, NB, DM), lambda i: (i, 0, 0)),
            pl.BlockSpec((1, NB, 3 * E), lambda i: (i, 0, 0)),
            pl.BlockSpec((NB, M, PREDH + 1), lambda i: (i, 0, 0)),
            pl.BlockSpec((NB, M, 1), lambda i: (i, 0, 0)),
        ],
        out_shape=[
            jax.ShapeDtypeStruct((grid, NB, DM), jnp.float32),
            jax.ShapeDtypeStruct((grid, NB, 3 * E), jnp.float32),
            jax.ShapeDtypeStruct((B, M, PREDH + 1), jnp.float32),
            jax.ShapeDtypeStruct((B, M, 1), jnp.float32),
        ],
        compiler_params=pltpu.CompilerParams(
            dimension_semantics=("arbitrary",)),
    )(x, mask3, maskr, p['revin_w'], p['revin_b'], pw, pepb,
      wcs[0], cbs[0], wcs[1], cbs[1], wcs[2], cbs[2],
      p['enc_ln_g'][None, :], p['enc_ln_b'][None, :],
      p['pool_W'], p['pool_b'][None, :], pos_t, tw,
      p['out_ln_g'][None, :], p['out_ln_b'][None, :], hwlf, hwt, hb, gw, gb)
    g = g3.reshape(B, DM)
    logits = logits3.reshape(B, 3 * E)

    pred, fail, rca3 = pl.pallas_call(
        _moe_body,
        out_shape=[
            jax.ShapeDtypeStruct((B, M, PREDH), jnp.float32),
            jax.ShapeDtypeStruct((B, FAILH), jnp.float32),
            jax.ShapeDtypeStruct((B, M, 1), jnp.float32),
        ],
    )(g, logits, head_part, sd, last3, w1cat, b1cat, w2stack, b2mat, expand,
      pwm, p['pred_b'][None, :], p['fail_W'], p['fail_b'][None, :], rwm)

    return pred, fail, rca3[..., 0]


# speedup vs baseline: 1.8171x; 1.8171x over previous
import math

import numpy as np
import jax
import jax.numpy as jnp
from jax.experimental import pallas as pl
from jax.experimental.pallas import tpu as pltpu

B, M, L = 128, 32, 256
PROJ, CNN_OUT, DM, NH, NLAYERS = 32, 32, 64, 4, 2
E, TOPK, EH, MOE_OUT = 64, 2, 128, 64
PREDH, FAILH = 8, 3
K = 3
DILS = (1, 2, 4)
NB = 4
NBM = NB * M
R = NBM * L


def _pe(length, d):
    pe = np.zeros((length, d), dtype=np.float32)
    pos = np.arange(length, dtype=np.float32)[:, None]
    div = np.exp(np.arange(0, d, 2, dtype=np.float32) * (-math.log(10000.0) / d))
    pe[:, 0::2] = np.sin(pos * div)
    pe[:, 1::2] = np.cos(pos * div)
    return pe


def _lnk(x, g, b, eps=1e-5):
    mu = jnp.mean(x, -1, keepdims=True)
    var = jnp.mean((x - mu) ** 2, -1, keepdims=True)
    return (x - mu) * jax.lax.rsqrt(var + eps) * g + b


def _shift_rows(h, s, iota_l):
    if s == 0:
        return h
    top = jnp.zeros((s, h.shape[1]), h.dtype)
    sh = jnp.concatenate([top, h[: h.shape[0] - s, :]], axis=0)
    return jnp.where(iota_l >= s, sh, 0.0)


def _encoder_body(x_ref, mask_ref, maskr_ref, mb_ref, bd_ref, revw_ref,
                  revb_ref, pw_ref,
                  pepb_ref, wc0_ref, cb0_ref, wc1_ref, cb1_ref, wc2_ref,
                  cb2_ref, encg_ref, encb_ref, poolw_ref, poolb_ref, pos_ref,
                  tw_ref, outg_ref, outb_ref, hwlf_ref, hwt_ref, hb_ref,
                  gw_ref, gb_ref,
                  g_ref, logits_ref, head_ref, sd_ref):
    x3 = x_ref[...]
    mu = jnp.mean(x3, -1, keepdims=True)
    var = jnp.mean((x3 - mu) ** 2, -1, keepdims=True)
    sd = jnp.sqrt(var + 1e-5)
    sd_ref[...] = sd
    xn3 = (x3 - mu) / sd * revw_ref[...] + revb_ref[...]
    xn2 = xn3.reshape(NBM, L)

    h0 = xn2[:, :, None] * pw_ref[...][None, :, :] + pepb_ref[...][None, :, :]
    h = h0.reshape(R, PROJ).astype(jnp.bfloat16)
    iota_l = jax.lax.broadcasted_iota(jnp.int32, (R, 1), 0) % L

    for wc_ref, cb_ref, dil in ((wc0_ref, cb0_ref, 1), (wc1_ref, cb1_ref, 2),
                                (wc2_ref, cb2_ref, 4)):
        a2 = _shift_rows(h, 2 * dil, iota_l)
        a1 = _shift_rows(h, dil, iota_l)
        cat = jnp.concatenate([a2, a1, h], axis=1)
        h = jnp.maximum(jnp.dot(cat, wc_ref[...],
                                preferred_element_type=jnp.float32)
                        + cb_ref[...], 0.0).astype(jnp.bfloat16)

    h = _lnk(h.astype(jnp.float32), encg_ref[...], encb_ref[...])
    h3 = h.reshape(NBM, L, CNN_OUT)
    maskr = maskr_ref[...]
    pooled = jnp.mean(h3, axis=1) * maskr
    lastf = h3[:, L - 1, :] * maskr

    ts = (jnp.dot(pooled, poolw_ref[...], preferred_element_type=jnp.float32)
          + poolb_ref[...] + pos_ref[...])

    tw = tw_ref[...]
    maskv = mask_ref[0]
    bias = bd_ref[...] + mb_ref[0]
    off = 0
    dh = DM // NH
    for _ in range(NLAYERS):
        ln1g = tw[off + 0:off + 1, :DM]; ln1b = tw[off + 1:off + 2, :DM]
        wq = tw[off + 2:off + 2 + DM, :DM];   bq = tw[off + 66:off + 67, :DM]
        wk = tw[off + 67:off + 67 + DM, :DM]; bk = tw[off + 131:off + 132, :DM]
        wv = tw[off + 132:off + 132 + DM, :DM]; bv = tw[off + 196:off + 197, :DM]
        wo = tw[off + 197:off + 197 + DM, :DM]; bo = tw[off + 261:off + 262, :DM]
        ln2g = tw[off + 262:off + 263, :DM]; ln2b = tw[off + 263:off + 264, :DM]
        ff1w = tw[off + 264:off + 264 + DM, :]
        ff1b = tw[off + 328:off + 329, :]
        ff2w = tw[off + 329:off + 329 + 2 * DM, :DM]
        ff2b = tw[off + 457:off + 458, :DM]
        off += 458

        a = _lnk(ts, ln1g, ln1b)
        q = jnp.dot(a, wq, preferred_element_type=jnp.float32) + bq
        k = jnp.dot(a, wk, preferred_element_type=jnp.float32) + bk
        v = jnp.dot(a, wv, preferred_element_type=jnp.float32) + bv
        ohs = []
        for hd in range(NH):
            sl = slice(hd * dh, (hd + 1) * dh)
            sc = jax.lax.dot_general(
                q[:, sl], k[:, sl], (((1,), (1,)), ((), ())),
                preferred_element_type=jnp.float32) * (1.0 / math.sqrt(dh))
            sc = sc + bias
            pa = jax.nn.softmax(sc, axis=-1)
            ohs.append(jnp.dot(pa, v[:, sl],
                               preferred_element_type=jnp.float32))
        o = jnp.dot(jnp.concatenate(ohs, axis=1), wo,
                    preferred_element_type=jnp.float32) + bo
        ts = ts + o
        hh = _lnk(ts, ln2g, ln2b)
        ts = ts + jnp.dot(
            jnp.maximum(jnp.dot(hh, ff1w,
                                preferred_element_type=jnp.float32) + ff1b,
                        0.0),
            ff2w, preferred_element_type=jnp.float32) + ff2b

    tl = _lnk(ts, outg_ref[...], outb_ref[...]) * maskr
    cnt = jnp.maximum(jnp.sum(maskv, axis=1, keepdims=True), 1.0)
    g = jnp.sum(tl.reshape(NB, M, DM), axis=1) / cnt
    g_ref[...] = g.reshape(1, NB, DM)
    logits_ref[...] = (jnp.dot(g, gw_ref[...],
                               preferred_element_type=jnp.float32)
                       + gb_ref[...]).reshape(1, NB, 3 * E)
    hp = (jnp.dot(lastf, hwlf_ref[...], preferred_element_type=jnp.float32)
          + jnp.dot(tl, hwt_ref[...], preferred_element_type=jnp.float32)
          + hb_ref[...])
    head_ref[...] = hp.reshape(NB, M, PREDH + 1)


def _gate_weights(v, iota64):
    m1 = jnp.max(v, -1, keepdims=True)
    i1 = jnp.min(jnp.where(v == m1, iota64, E), -1, keepdims=True)
    sel1 = iota64 == i1
    v2 = jnp.where(sel1, -1e30, v)
    m2 = jnp.max(v2, -1, keepdims=True)
    i2 = jnp.min(jnp.where(v2 == m2, iota64, E), -1, keepdims=True)
    sel2 = iota64 == i2
    g1 = 1.0 / (1.0 + jnp.exp(m2 - m1))
    g2 = 1.0 - g1
    return jnp.where(sel1, g1, 0.0) + jnp.where(sel2, g2, 0.0)


def _moe_body(g_ref, logits_ref, head_ref, sd_ref, last_ref, w1_ref, b1_ref,
              w2_ref, b2_ref, exp_ref, pwm_ref, pb_ref, failw_ref, failb_ref,
              rwm_ref, pred_ref, fail_ref, rca_ref):
    gg = g_ref[...]
    h1 = jnp.maximum(jnp.dot(gg, w1_ref[...],
                             preferred_element_type=jnp.float32)
                     + b1_ref[...], 0.0)
    lg = logits_ref[...]
    iota64 = jax.lax.broadcasted_iota(jnp.int32, (B, E), 1)
    moes = []
    for gi in range(3):
        w = _gate_weights(lg[:, gi * E:(gi + 1) * E], iota64)
        wexp = jnp.dot(w, exp_ref[...],
                       preferred_element_type=jnp.float32)
        moe = (jnp.dot(h1 * wexp, w2_ref[...],
                       preferred_element_type=jnp.float32)
               + jnp.dot(w, b2_ref[...], preferred_element_type=jnp.float32))
        moes.append(moe)
    moe_f, moe_l, moe_r = moes

    hp = head_ref[...]
    predm = jnp.dot(moe_f, pwm_ref[...],
                    preferred_element_type=jnp.float32) + pb_ref[...]
    pred_delta = hp[:, :, 0:PREDH] + predm[:, None, :]
    pred_ref[...] = pred_delta * sd_ref[...] + last_ref[...]
    fail_ref[...] = (jnp.dot(moe_l, failw_ref[...],
                             preferred_element_type=jnp.float32)
                     + failb_ref[...])
    rcam = jnp.dot(moe_r, rwm_ref[...], preferred_element_type=jnp.float32)
    rca_ref[...] = hp[:, :, PREDH:PREDH + 1] + rcam[:, None, :]


def _const_spec(shape):
    nd = len(shape)
    return pl.BlockSpec(shape, lambda i, _n=nd: (0,) * _n)


@jax.jit
def kernel(x_features_orig_scale, sensor_mask, last_known_values_orig, params):
    p = params
    x = x_features_orig_scale
    mask = sensor_mask

    pepb = jnp.asarray(_pe(L, PROJ)) + p['proj_b'][None, :]
    pw = jnp.broadcast_to(p['proj_W'], (L, PROJ))
    wcs = [p['conv%d_W' % i].transpose(2, 1, 0).reshape(K * PROJ, -1)
           .astype(jnp.bfloat16) for i in range(3)]
    cbs = [p['conv%d_b' % i][None, :] for i in range(3)]
    maskr = mask.reshape(B * M, 1)
    pos_t = jnp.tile(p['pos'][0], (NB, 1))
    bd = jnp.asarray(np.where(
        (np.arange(NBM)[:, None] // M) == (np.arange(NBM)[None, :] // M),
        np.float32(0.0), np.float32(-1e9)))
    mb = ((mask - 1.0) * 1e9).reshape(B // NB, 1, NBM)

    tw_rows = []
    for l in range(NLAYERS):
        pr = 't%d_' % l
        def pad(a):
            a = a if a.ndim == 2 else a[None, :]
            return jnp.pad(a, ((0, 0), (0, 2 * DM - a.shape[1])))
        tw_rows += [pad(p[pr + 'ln1_g']), pad(p[pr + 'ln1_b']),
                    pad(p[pr + 'Wq']), pad(p[pr + 'bq']),
                    pad(p[pr + 'Wk']), pad(p[pr + 'bk']),
                    pad(p[pr + 'Wv']), pad(p[pr + 'bv']),
                    pad(p[pr + 'Wo']), pad(p[pr + 'bo']),
                    pad(p[pr + 'ln2_g']), pad(p[pr + 'ln2_b']),
                    p[pr + 'ff1_W'], p[pr + 'ff1_b'][None, :],
                    pad(p[pr + 'ff2_W']), pad(p[pr + 'ff2_b'])]
    tw = jnp.concatenate(tw_rows, axis=0)

    hwlf = jnp.concatenate([p['pred_W'][:CNN_OUT], p['rca_W'][:CNN_OUT]], 1)
    hwt = jnp.concatenate([p['pred_W'][CNN_OUT:CNN_OUT + DM],
                           p['rca_W'][CNN_OUT:CNN_OUT + DM]], 1)
    hb = jnp.concatenate([p['pred_b'], p['rca_b']])[None, :]
    gw = jnp.concatenate([p['gf_W'], p['gl_W'], p['gr_W']], axis=1)
    gb = jnp.concatenate([p['gf_b'], p['gl_b'], p['gr_b']])[None, :]

    w1cat = p['e_W1'].transpose(1, 0, 2).reshape(DM, E * EH)
    b1cat = p['e_b1'].reshape(1, E * EH)
    w2stack = p['e_W2'].reshape(E * EH, MOE_OUT)
    b2mat = p['e_b2']
    expand = jnp.asarray(np.repeat(np.eye(E, dtype=np.float32), EH, axis=1))
    pwm = p['pred_W'][CNN_OUT + DM:]
    rwm = p['rca_W'][CNN_OUT + DM:]
    last3 = last_known_values_orig[:, :, None]

    grid = B // NB
    mask3 = mask.reshape(grid, NB, M)
    g3, logits3, head_part, sd = pl.pallas_call(
        _encoder_body,
        grid=(grid,),
        in_specs=[
            pl.BlockSpec((NB, M, L), lambda i: (i, 0, 0)),
            pl.BlockSpec((1, NB, M), lambda i: (i, 0, 0)),
            pl.BlockSpec((NBM, 1), lambda i: (i, 0)),
            pl.BlockSpec((1, 1, NBM), lambda i: (i, 0, 0)),
            _const_spec((NBM, NBM)),
            _const_spec((1, M, 1)), _const_spec((1, M, 1)),
            _const_spec((L, PROJ)), _const_spec((L, PROJ)),
            _const_spec((K * PROJ, PROJ)), _const_spec((1, PROJ)),
            _const_spec((K * PROJ, PROJ)), _const_spec((1, PROJ)),
            _const_spec((K * PROJ, CNN_OUT)), _const_spec((1, CNN_OUT)),
            _const_spec((1, CNN_OUT)), _const_spec((1, CNN_OUT)),
            _const_spec((CNN_OUT, DM)), _const_spec((1, DM)),
            _const_spec((NBM, DM)),
            _const_spec(tuple(tw.shape)),
            _const_spec((1, DM)), _const_spec((1, DM)),
            _const_spec((CNN_OUT, PREDH + 1)), _const_spec((DM, PREDH + 1)),
            _const_spec((1, PREDH + 1)),
            _const_spec((DM, 3 * E)), _const_spec((1, 3 * E)),
        ],
        out_specs=[
            pl.BlockSpec((1, NB, DM), lambda i: (i, 0, 0)),
            pl.BlockSpec((1, NB, 3 * E), lambda i: (i, 0, 0)),
            pl.BlockSpec((NB, M, PREDH + 1), lambda i: (i, 0, 0)),
            pl.BlockSpec((NB, M, 1), lambda i: (i, 0, 0)),
        ],
        out_shape=[
            jax.ShapeDtypeStruct((grid, NB, DM), jnp.float32),
            jax.ShapeDtypeStruct((grid, NB, 3 * E), jnp.float32),
            jax.ShapeDtypeStruct((B, M, PREDH + 1), jnp.float32),
            jax.ShapeDtypeStruct((B, M, 1), jnp.float32),
        ],
        compiler_params=pltpu.CompilerParams(
            dimension_semantics=("parallel",)),
    )(x, mask3, maskr, mb, bd, p['revin_w'], p['revin_b'], pw, pepb,
      wcs[0], cbs[0], wcs[1], cbs[1], wcs[2], cbs[2],
      p['enc_ln_g'][None, :], p['enc_ln_b'][None, :],
      p['pool_W'], p['pool_b'][None, :], pos_t, tw,
      p['out_ln_g'][None, :], p['out_ln_b'][None, :], hwlf, hwt, hb, gw, gb)
    g = g3.reshape(B, DM)
    logits = logits3.reshape(B, 3 * E)

    pred, fail, rca3 = pl.pallas_call(
        _moe_body,
        out_shape=[
            jax.ShapeDtypeStruct((B, M, PREDH), jnp.float32),
            jax.ShapeDtypeStruct((B, FAILH), jnp.float32),
            jax.ShapeDtypeStruct((B, M, 1), jnp.float32),
        ],
    )(g, logits, head_part, sd, last3, w1cat, b1cat, w2stack, b2mat, expand,
      pwm, p['pred_b'][None, :], p['fail_W'], p['fail_b'][None, :], rwm)

    return pred, fail, rca3[..., 0]


# speedup vs baseline: 6.7420x; 3.7103x over previous
import dataclasses
import functools
import math

import numpy as np
import jax
import jax.numpy as jnp
from jax.experimental import pallas as pl
from jax.experimental.pallas import tpu as pltpu
from jax.experimental.pallas import tpu_sc as plsc

B, M, L = 128, 32, 256
PROJ, CNN_OUT, DM, NH, NLAYERS = 32, 32, 64, 4, 2
E, TOPK, EH, MOE_OUT = 64, 2, 128, 64
PREDH, FAILH = 8, 3
K = 3
DILS = (1, 2, 4)
NB = 8
NBM = NB * M
R = NBM * L


def _pe(length, d):
    pe = np.zeros((length, d), dtype=np.float32)
    pos = np.arange(length, dtype=np.float32)[:, None]
    div = np.exp(np.arange(0, d, 2, dtype=np.float32) * (-math.log(10000.0) / d))
    pe[:, 0::2] = np.sin(pos * div)
    pe[:, 1::2] = np.cos(pos * div)
    return pe


def _lnk(x, g, b, eps=1e-5):
    mu = jnp.mean(x, -1, keepdims=True)
    var = jnp.mean((x - mu) ** 2, -1, keepdims=True)
    return (x - mu) * jax.lax.rsqrt(var + eps) * g + b


def _encoder_body(xt_ref, mask_ref, maskr_ref, mt_ref, mb_ref, bd_ref,
                  revw_ref, revb_ref, pw_ref,
                  petb_ref, wc0_ref, cb0_ref, wc1_ref, cb1_ref, wc2_ref,
                  cb2_ref, encg_ref, encb_ref, poolw_ref, poolb_ref, pos_ref,
                  tw_ref, outg_ref, outb_ref, hwlf_ref, hwt_ref, hb_ref,
                  gw_ref, gb_ref,
                  g_ref, logits_ref, head_ref, sd_ref):
    xt = xt_ref[...]
    mu = jnp.mean(xt, axis=0, keepdims=True)
    var = jnp.mean((xt - mu) ** 2, axis=0, keepdims=True)
    sd = jnp.sqrt(var + 1e-5)
    sd_ref[...] = sd.reshape(1, 1, NBM)
    xn = (xt - mu) / sd * revw_ref[...] + revb_ref[...]

    h0 = (xn[None, :, :] * pw_ref[...][:, :, None]
          + petb_ref[...][:, :, None])
    h = h0.reshape(PROJ, R).astype(jnp.bfloat16)
    hf = None
    for wc_ref, cb_ref, dil in ((wc0_ref, cb0_ref, 1), (wc1_ref, cb1_ref, 2),
                                (wc2_ref, cb2_ref, 4)):
        s1, s2 = dil * NBM, 2 * dil * NBM
        sh2 = jnp.concatenate(
            [jnp.zeros((PROJ, s2), jnp.bfloat16), h[:, :R - s2]], axis=1)
        sh1 = jnp.concatenate(
            [jnp.zeros((PROJ, s1), jnp.bfloat16), h[:, :R - s1]], axis=1)
        cat = jnp.concatenate([sh2, sh1, h], axis=0)
        hf = jnp.maximum(jnp.dot(wc_ref[...], cat,
                                 preferred_element_type=jnp.float32)
                         + cb_ref[...], 0.0)
        h = hf.astype(jnp.bfloat16)

    mu_c = jnp.mean(hf, axis=0, keepdims=True)
    var_c = jnp.mean((hf - mu_c) ** 2, axis=0, keepdims=True)
    hn = ((hf - mu_c) * jax.lax.rsqrt(var_c + 1e-5) * encg_ref[...]
          + encb_ref[...])
    h3 = hn.reshape(CNN_OUT, L, NBM)
    mt = mt_ref[0]
    pooled_cm = jnp.mean(h3, axis=1) * mt
    lastf_cm = h3[:, L - 1, :] * mt
    pooled = pooled_cm.T
    lastf = lastf_cm.T
    maskr = maskr_ref[...]

    ts = (jnp.dot(pooled, poolw_ref[...], preferred_element_type=jnp.float32)
          + poolb_ref[...] + pos_ref[...])

    tw = tw_ref[...]
    maskv = mask_ref[0]
    bias = bd_ref[...] + mb_ref[0]
    off = 0
    dh = DM // NH
    for _ in range(NLAYERS):
        ln1g = tw[off + 0:off + 1, :DM]; ln1b = tw[off + 1:off + 2, :DM]
        wq = tw[off + 2:off + 2 + DM, :DM];   bq = tw[off + 66:off + 67, :DM]
        wk = tw[off + 67:off + 67 + DM, :DM]; bk = tw[off + 131:off + 132, :DM]
        wv = tw[off + 132:off + 132 + DM, :DM]; bv = tw[off + 196:off + 197, :DM]
        wo = tw[off + 197:off + 197 + DM, :DM]; bo = tw[off + 261:off + 262, :DM]
        ln2g = tw[off + 262:off + 263, :DM]; ln2b = tw[off + 263:off + 264, :DM]
        ff1w = tw[off + 264:off + 264 + DM, :]
        ff1b = tw[off + 328:off + 329, :]
        ff2w = tw[off + 329:off + 329 + 2 * DM, :DM]
        ff2b = tw[off + 457:off + 458, :DM]
        off += 458

        a = _lnk(ts, ln1g, ln1b)
        q = jnp.dot(a, wq, preferred_element_type=jnp.float32) + bq
        k = jnp.dot(a, wk, preferred_element_type=jnp.float32) + bk
        v = jnp.dot(a, wv, preferred_element_type=jnp.float32) + bv
        ohs = []
        for hd in range(NH):
            sl = slice(hd * dh, (hd + 1) * dh)
            sc = jax.lax.dot_general(
                q[:, sl], k[:, sl], (((1,), (1,)), ((), ())),
                preferred_element_type=jnp.float32) * (1.0 / math.sqrt(dh))
            sc = sc + bias
            pa = jax.nn.softmax(sc, axis=-1)
            ohs.append(jnp.dot(pa, v[:, sl],
                               preferred_element_type=jnp.float32))
        o = jnp.dot(jnp.concatenate(ohs, axis=1), wo,
                    preferred_element_type=jnp.float32) + bo
        ts = ts + o
        hh = _lnk(ts, ln2g, ln2b)
        ts = ts + jnp.dot(
            jnp.maximum(jnp.dot(hh, ff1w,
                                preferred_element_type=jnp.float32) + ff1b,
                        0.0),
            ff2w, preferred_element_type=jnp.float32) + ff2b

    tl = _lnk(ts, outg_ref[...], outb_ref[...]) * maskr
    cnt = jnp.maximum(jnp.sum(maskv, axis=1, keepdims=True), 1.0)
    g = jnp.sum(tl.reshape(NB, M, DM), axis=1) / cnt
    g_ref[...] = g.reshape(1, NB, DM)
    logits_ref[...] = (jnp.dot(g, gw_ref[...],
                               preferred_element_type=jnp.float32)
                       + gb_ref[...]).reshape(1, NB, 3 * E)
    hp = (jnp.dot(lastf, hwlf_ref[...], preferred_element_type=jnp.float32)
          + jnp.dot(tl, hwt_ref[...], preferred_element_type=jnp.float32)
          + hb_ref[...])
    head_ref[...] = hp.reshape(NB, M, PREDH + 1)


_SC_NC, _SC_NS, _SC_LN = 2, 16, 16
_SC_NW = _SC_NC * _SC_NS
_SC_RPW = B // _SC_NW


def _routing_sc(logits):
    mesh = plsc.VectorSubcoreMesh(core_axis_name="c", subcore_axis_name="s")
    cp = pltpu.CompilerParams()
    if "needs_layout_passes" in pltpu.CompilerParams.__dataclass_fields__:
        cp = dataclasses.replace(cp, needs_layout_passes=False)

    @functools.partial(
        pl.kernel,
        out_type=jax.ShapeDtypeStruct((B, 3 * E), jnp.float32),
        mesh=mesh,
        compiler_params=cp,
        scratch_types=[
            pltpu.VMEM((_SC_RPW, 3 * E), jnp.float32),
            pltpu.VMEM((_SC_RPW, 3 * E), jnp.float32),
            pltpu.SemaphoreType.DMA,
        ],
    )
    def rk(lg_hbm, w_hbm, lg_v, w_v, sem):
        wid = jax.lax.axis_index("s") * _SC_NC + jax.lax.axis_index("c")
        base = wid * _SC_RPW
        pltpu.async_copy(lg_hbm.at[pl.ds(base, _SC_RPW)], lg_v, sem).wait()
        iota = jax.lax.iota(jnp.int32, _SC_LN)
        for r in range(_SC_RPW):
            for gi in range(3):
                vs = [lg_v[r, pl.ds(gi * E + j * _SC_LN, _SC_LN)]
                      for j in range(4)]
                poss = [iota + j * _SC_LN for j in range(4)]
                m1 = jnp.max(jnp.maximum(jnp.maximum(vs[0], vs[1]),
                                         jnp.maximum(vs[2], vs[3])))
                cands = [jnp.where(vs[j] == m1, poss[j], E) for j in range(4)]
                i1 = jnp.min(jnp.minimum(jnp.minimum(cands[0], cands[1]),
                                         jnp.minimum(cands[2], cands[3])))
                v2s = [jnp.where(poss[j] == i1, -1e30, vs[j])
                       for j in range(4)]
                m2 = jnp.max(jnp.maximum(jnp.maximum(v2s[0], v2s[1]),
                                         jnp.maximum(v2s[2], v2s[3])))
                c2s = [jnp.where(v2s[j] == m2, poss[j], E) for j in range(4)]
                i2 = jnp.min(jnp.minimum(jnp.minimum(c2s[0], c2s[1]),
                                         jnp.minimum(c2s[2], c2s[3])))
                dv = jnp.full((_SC_LN,), m2 - m1, jnp.float32)
                g1 = 1.0 / (1.0 + jnp.exp(dv))
                g2 = 1.0 - g1
                for j in range(4):
                    wj = (jnp.where(poss[j] == i1, g1, 0.0)
                          + jnp.where(poss[j] == i2, g2, 0.0))
                    w_v[r, pl.ds(gi * E + j * _SC_LN, _SC_LN)] = wj
        pltpu.async_copy(w_v, w_hbm.at[pl.ds(base, _SC_RPW)], sem).wait()

    return rk(logits)


def _h1_body(g_ref, w1_ref, b1_ref, h1_ref):
    h1_ref[...] = jnp.maximum(
        jnp.dot(g_ref[...], w1_ref[...], preferred_element_type=jnp.float32)
        + b1_ref[...], 0.0)


def _moe_body(h1_ref, wts_ref, head_ref, sd_ref, last_ref,
              w2_ref, b2_ref, exp_ref, pwm_ref, pb_ref, failw_ref, failb_ref,
              rwm_ref, pred_ref, fail_ref, rca_ref):
    h1 = h1_ref[...]
    wts = wts_ref[...]
    w2b = w2_ref[...]
    hws, wss = [], []
    for gi in range(3):
        w = wts[:, gi * E:(gi + 1) * E]
        wexp = jnp.dot(w.astype(jnp.bfloat16), exp_ref[...],
                       preferred_element_type=jnp.float32)
        hws.append((h1 * wexp).astype(jnp.bfloat16))
        wss.append(w)
    hw3 = jnp.concatenate(hws, axis=0)
    w3 = jnp.concatenate(wss, axis=0)
    moe3 = (jnp.dot(hw3, w2b, preferred_element_type=jnp.float32)
            + jnp.dot(w3, b2_ref[...], preferred_element_type=jnp.float32))
    moe_f, moe_l, moe_r = moe3[:B], moe3[B:2 * B], moe3[2 * B:]

    hp = head_ref[...]
    predm = jnp.dot(moe_f, pwm_ref[...],
                    preferred_element_type=jnp.float32) + pb_ref[...]
    pred_delta = hp[:, :, 0:PREDH] + predm[:, None, :]
    pred_ref[...] = pred_delta * sd_ref[...] + last_ref[...]
    fail_ref[...] = (jnp.dot(moe_l, failw_ref[...],
                             preferred_element_type=jnp.float32)
                     + failb_ref[...])
    rcam = jnp.dot(moe_r, rwm_ref[...], preferred_element_type=jnp.float32)
    rca_ref[...] = hp[:, :, PREDH:PREDH + 1] + rcam[:, None, :]


def _const_spec(shape):
    nd = len(shape)
    return pl.BlockSpec(shape, lambda i, _n=nd: (0,) * _n)


@jax.jit
def kernel(x_features_orig_scale, sensor_mask, last_known_values_orig, params):
    p = params
    x = x_features_orig_scale
    mask = sensor_mask

    xt = x.transpose(2, 0, 1).reshape(L, B * M)
    petb = (jnp.asarray(_pe(L, PROJ)) + p['proj_b'][None, :]).T
    pwc = p['proj_W'].reshape(PROJ, 1)
    wcs = [p['conv%d_W' % i].transpose(0, 2, 1).reshape(-1, K * PROJ)
           .astype(jnp.bfloat16) for i in range(3)]
    cbs = [p['conv%d_b' % i][:, None] for i in range(3)]
    revw_t = jnp.tile(p['revin_w'][0, :, 0], NB)[None, :]
    revb_t = jnp.tile(p['revin_b'][0, :, 0], NB)[None, :]
    maskr = mask.reshape(B * M, 1)
    mask_t = mask.reshape(B // NB, 1, NBM)
    pos_t = jnp.tile(p['pos'][0], (NB, 1))
    bd = jnp.asarray(np.where(
        (np.arange(NBM)[:, None] // M) == (np.arange(NBM)[None, :] // M),
        np.float32(0.0), np.float32(-1e9)))
    mb = ((mask - 1.0) * 1e9).reshape(B // NB, 1, NBM)

    tw_rows = []
    for l in range(NLAYERS):
        pr = 't%d_' % l
        def pad(a):
            a = a if a.ndim == 2 else a[None, :]
            return jnp.pad(a, ((0, 0), (0, 2 * DM - a.shape[1])))
        tw_rows += [pad(p[pr + 'ln1_g']), pad(p[pr + 'ln1_b']),
                    pad(p[pr + 'Wq']), pad(p[pr + 'bq']),
                    pad(p[pr + 'Wk']), pad(p[pr + 'bk']),
                    pad(p[pr + 'Wv']), pad(p[pr + 'bv']),
                    pad(p[pr + 'Wo']), pad(p[pr + 'bo']),
                    pad(p[pr + 'ln2_g']), pad(p[pr + 'ln2_b']),
                    p[pr + 'ff1_W'], p[pr + 'ff1_b'][None, :],
                    pad(p[pr + 'ff2_W']), pad(p[pr + 'ff2_b'])]
    tw = jnp.concatenate(tw_rows, axis=0)

    hwlf = jnp.concatenate([p['pred_W'][:CNN_OUT], p['rca_W'][:CNN_OUT]], 1)
    hwt = jnp.concatenate([p['pred_W'][CNN_OUT:CNN_OUT + DM],
                           p['rca_W'][CNN_OUT:CNN_OUT + DM]], 1)
    hb = jnp.concatenate([p['pred_b'], p['rca_b']])[None, :]
    gw = jnp.concatenate([p['gf_W'], p['gl_W'], p['gr_W']], axis=1)
    gb = jnp.concatenate([p['gf_b'], p['gl_b'], p['gr_b']])[None, :]

    w1cat = p['e_W1'].transpose(1, 0, 2).reshape(DM, E * EH)
    b1cat = p['e_b1'].reshape(1, E * EH)
    w2stack = p['e_W2'].reshape(E * EH, MOE_OUT).astype(jnp.bfloat16)
    b2mat = p['e_b2']
    expand = jnp.asarray(np.repeat(np.eye(E, dtype=np.float32), EH,
                                   axis=1)).astype(jnp.bfloat16)
    pwm = p['pred_W'][CNN_OUT + DM:]
    rwm = p['rca_W'][CNN_OUT + DM:]
    last3 = last_known_values_orig[:, :, None]

    grid = B // NB
    mask3 = mask.reshape(grid, NB, M)
    g3, logits3, head_part, sd3 = pl.pallas_call(
        _encoder_body,
        grid=(grid,),
        in_specs=[
            pl.BlockSpec((L, NBM), lambda i: (0, i)),
            pl.BlockSpec((1, NB, M), lambda i: (i, 0, 0)),
            pl.BlockSpec((NBM, 1), lambda i: (i, 0)),
            pl.BlockSpec((1, 1, NBM), lambda i: (i, 0, 0)),
            pl.BlockSpec((1, 1, NBM), lambda i: (i, 0, 0)),
            _const_spec((NBM, NBM)),
            _const_spec((1, NBM)), _const_spec((1, NBM)),
            _const_spec((PROJ, 1)), _const_spec((PROJ, L)),
            _const_spec((PROJ, K * PROJ)), _const_spec((PROJ, 1)),
            _const_spec((PROJ, K * PROJ)), _const_spec((PROJ, 1)),
            _const_spec((CNN_OUT, K * PROJ)), _const_spec((CNN_OUT, 1)),
            _const_spec((CNN_OUT, 1)), _const_spec((CNN_OUT, 1)),
            _const_spec((CNN_OUT, DM)), _const_spec((1, DM)),
            _const_spec((NBM, DM)),
            _const_spec(tuple(tw.shape)),
            _const_spec((1, DM)), _const_spec((1, DM)),
            _const_spec((CNN_OUT, PREDH + 1)), _const_spec((DM, PREDH + 1)),
            _const_spec((1, PREDH + 1)),
            _const_spec((DM, 3 * E)), _const_spec((1, 3 * E)),
        ],
        out_specs=[
            pl.BlockSpec((1, NB, DM), lambda i: (i, 0, 0)),
            pl.BlockSpec((1, NB, 3 * E), lambda i: (i, 0, 0)),
            pl.BlockSpec((NB, M, PREDH + 1), lambda i: (i, 0, 0)),
            pl.BlockSpec((1, 1, NBM), lambda i: (i, 0, 0)),
        ],
        out_shape=[
            jax.ShapeDtypeStruct((grid, NB, DM), jnp.float32),
            jax.ShapeDtypeStruct((grid, NB, 3 * E), jnp.float32),
            jax.ShapeDtypeStruct((B, M, PREDH + 1), jnp.float32),
            jax.ShapeDtypeStruct((grid, 1, NBM), jnp.float32),
        ],
        compiler_params=pltpu.CompilerParams(
            dimension_semantics=("parallel",)),
    )(xt, mask3, maskr, mask_t, mb, bd, revw_t, revb_t, pwc, petb,
      wcs[0], cbs[0], wcs[1], cbs[1], wcs[2], cbs[2],
      p['enc_ln_g'][:, None], p['enc_ln_b'][:, None],
      p['pool_W'], p['pool_b'][None, :], pos_t, tw,
      p['out_ln_g'][None, :], p['out_ln_b'][None, :], hwlf, hwt, hb, gw, gb)
    g = g3.reshape(B, DM)
    logits = logits3.reshape(B, 3 * E)
    sd = sd3.reshape(B, M)[:, :, None]

    wts = _routing_sc(logits)
    h1 = pl.pallas_call(
        _h1_body,
        out_shape=jax.ShapeDtypeStruct((B, E * EH), jnp.float32),
    )(g, w1cat, b1cat)

    pred, fail, rca3 = pl.pallas_call(
        _moe_body,
        out_shape=[
            jax.ShapeDtypeStruct((B, M, PREDH), jnp.float32),
            jax.ShapeDtypeStruct((B, FAILH), jnp.float32),
            jax.ShapeDtypeStruct((B, M, 1), jnp.float32),
        ],
    )(h1, wts, head_part, sd, last3, w2stack, b2mat, expand,
      pwm, p['pred_b'][None, :], p['fail_W'], p['fail_b'][None, :], rwm)

    return pred, fail, rca3[..., 0]
